# Initial kernel scaffold; baseline (speedup 1.0000x reference)
import jax
import jax.numpy as jnp
from jax.experimental import pallas as pl


def kernel(x, edge_index, W1, b1, W2, b2, W21, b21, W3, b3):
    raise NotImplementedError("write your pallas kernel here")


# speedup vs baseline: 8.3242x; 8.3242x over previous
import functools

import jax
import jax.numpy as jnp
from jax import lax
from jax.experimental import pallas as pl
from jax.experimental.pallas import tpu as pltpu
from jax.experimental.pallas import tpu_sc as plsc

N = 100000
E = 1600000
FEATS = 4
HID = 32
HALF = 16

N_P = 100352
N_TRASH = 128
N_ACC = N_P + N_TRASH

CHUNK = 128
NSC = 2
NTILE = 16
E_PAD = 1601536

EPT = E_PAD // NTILE
NCHUNK = EPT // CHUNK
EPT_DEG = E_PAD // (NSC * NTILE)
NCHUNK_DEG = EPT_DEG // CHUNK

RPT = N_P // NTILE

R = 1024
NB = N_P // R

_MESH = plsc.VectorSubcoreMesh(core_axis_name="c", subcore_axis_name="s")
_SC_PARAMS = pltpu.CompilerParams(use_tc_tiling_on_sc=False)


@functools.partial(
    pl.kernel,
    out_type=jax.ShapeDtypeStruct((NSC, N_P), jnp.float32),
    mesh=_MESH,
    scratch_types=[
        pltpu.VMEM((CHUNK,), jnp.int32),
        pltpu.VMEM((CHUNK,), jnp.float32),
        pltpu.VMEM_SHARED((N_ACC,), jnp.float32),
    ],
    compiler_params=_SC_PARAMS,
)
def _deg_kernel(dst_hbm, init_hbm, ones_hbm, out_hbm, idxbuf, onesbuf, acc):
    c = lax.axis_index("c")
    s = lax.axis_index("s")
    r0 = s * RPT
    pltpu.sync_copy(init_hbm.at[c, pl.ds(r0, RPT)], acc.at[pl.ds(r0, RPT)])
    pltpu.sync_copy(ones_hbm, onesbuf)
    plsc.subcore_barrier()
    base = c * (E_PAD // NSC) + s * EPT_DEG

    @pl.loop(0, NCHUNK_DEG)
    def _chunk(j):
        off = base + j * CHUNK
        pltpu.sync_copy(dst_hbm.at[pl.ds(off, CHUNK)], idxbuf)
        pltpu.sync_copy(onesbuf, acc.at[idxbuf], add=True)

    plsc.subcore_barrier()
    pltpu.sync_copy(acc.at[pl.ds(r0, RPT)], out_hbm.at[c, pl.ds(r0, RPT)])


@functools.partial(
    pl.kernel,
    out_type=jax.ShapeDtypeStruct((NSC, N_P, HALF), jnp.float32),
    mesh=_MESH,
    scratch_types=[
        pltpu.VMEM((CHUNK,), jnp.int32),
        pltpu.VMEM((CHUNK,), jnp.int32),
        pltpu.VMEM((CHUNK, HALF), jnp.float32),
        pltpu.VMEM_SHARED((N_ACC, HALF), jnp.float32),
        pltpu.SemaphoreType.DMA,
    ],
    compiler_params=_SC_PARAMS,
)
def _agg_kernel(g_hbm, src_hbm, dst_hbm, out_hbm, srcbuf, dstbuf, rows, acc, sem):
    c = lax.axis_index("c")
    s = lax.axis_index("s")
    gc = g_hbm.at[c]
    r0 = s * RPT
    pltpu.sync_copy(gc.at[pl.ds(r0, RPT)], acc.at[pl.ds(r0, RPT)])
    plsc.subcore_barrier()
    base = s * EPT

    @pl.loop(0, NCHUNK)
    def _chunk(j):
        off = base + j * CHUNK
        pltpu.sync_copy(src_hbm.at[pl.ds(off, CHUNK)], srcbuf)
        pltpu.sync_copy(dst_hbm.at[pl.ds(off, CHUNK)], dstbuf)
        pltpu.async_copy(gc.at[srcbuf], rows, sem).wait()
        pltpu.sync_copy(rows, acc.at[dstbuf], add=True)

    plsc.subcore_barrier()
    pltpu.sync_copy(acc.at[pl.ds(r0, RPT)], out_hbm.at[c, pl.ds(r0, RPT)])


def _tc0_body(deg_ref, x_ref, w_ref, d_ref, g_ref):
    dv = lax.rsqrt(deg_ref[0] + deg_ref[1])
    d_ref[...] = dv
    h = jnp.dot(x_ref[...], w_ref[0], preferred_element_type=jnp.float32)
    g_ref[0] = dv * h


def _tc_mid_body(acc_ref, d_ref, b_ref, w_ref, g_ref):
    dv = d_ref[...]
    a = jnp.concatenate([acc_ref[0], acc_ref[1]], axis=-1)
    xk = jnp.maximum(dv * a + b_ref[...], 0.0)
    g_ref[0] = dv * jnp.dot(xk, w_ref[0], preferred_element_type=jnp.float32)


def _tc_fin_body(acc_ref, d_ref, w_ref, b_ref, o_ref):
    dv = d_ref[...]
    a = jnp.concatenate([acc_ref[0], acc_ref[1]], axis=-1)
    o_ref[...] = dv * jnp.dot(a, w_ref[...], preferred_element_type=jnp.float32) + b_ref[...]


_tc0 = pl.pallas_call(
    _tc0_body,
    grid=(NSC, NB),
    in_specs=[
        pl.BlockSpec((NSC, R, 1), lambda c, i: (0, i, 0)),
        pl.BlockSpec((R, FEATS), lambda c, i: (i, 0)),
        pl.BlockSpec((1, FEATS, HALF), lambda c, i: (c, 0, 0)),
    ],
    out_specs=[
        pl.BlockSpec((R, 1), lambda c, i: (i, 0)),
        pl.BlockSpec((1, R, HALF), lambda c, i: (c, i, 0)),
    ],
    out_shape=[
        jax.ShapeDtypeStruct((N_P, 1), jnp.float32),
        jax.ShapeDtypeStruct((NSC, N_P, HALF), jnp.float32),
    ],
)

_tc_mid = pl.pallas_call(
    _tc_mid_body,
    grid=(NSC, NB),
    in_specs=[
        pl.BlockSpec((NSC, R, HALF), lambda c, i: (0, i, 0)),
        pl.BlockSpec((R, 1), lambda c, i: (i, 0)),
        pl.BlockSpec((1, HID), lambda c, i: (0, 0)),
        pl.BlockSpec((1, HID, HALF), lambda c, i: (c, 0, 0)),
    ],
    out_specs=pl.BlockSpec((1, R, HALF), lambda c, i: (c, i, 0)),
    out_shape=jax.ShapeDtypeStruct((NSC, N_P, HALF), jnp.float32),
)

_tc_fin = pl.pallas_call(
    _tc_fin_body,
    grid=(NB,),
    in_specs=[
        pl.BlockSpec((NSC, R, HALF), lambda i: (0, i, 0)),
        pl.BlockSpec((R, 1), lambda i: (i, 0)),
        pl.BlockSpec((HID, 1), lambda i: (0, 0)),
        pl.BlockSpec((1, 1), lambda i: (0, 0)),
    ],
    out_specs=pl.BlockSpec((R, 1), lambda i: (i, 0)),
    out_shape=jax.ShapeDtypeStruct((N_P, 1), jnp.float32),
)


def _split_w(w):
    return w.reshape(w.shape[0], NSC, HALF).transpose(1, 0, 2)


def kernel(x, edge_index, W1, b1, W2, b2, W21, b21, W3, b3):
    f32 = jnp.float32
    src = edge_index[0]
    dst = edge_index[1]
    pad = E_PAD - E
    src_p = jnp.concatenate([src, jnp.zeros((pad,), jnp.int32)])
    trash = N_P + (jnp.arange(pad, dtype=jnp.int32) % N_TRASH)
    dst_p = jnp.concatenate([dst, trash])

    x_p = jnp.zeros((N_P, FEATS), f32).at[:N].set(x)
    deg_init = jnp.stack([jnp.ones((N_P,), f32), jnp.zeros((N_P,), f32)])
    ones_c = jnp.ones((CHUNK,), f32)

    W1r = _split_w(W1)
    W2r = _split_w(W2)
    W21r = _split_w(W21)
    EYEr = _split_w(jnp.eye(HID, dtype=f32))

    deg = _deg_kernel(dst_p, deg_init, ones_c)
    d, g1 = _tc0(deg.reshape(NSC, N_P, 1), x_p, W1r)
    acc1 = _agg_kernel(g1, src_p, dst_p)
    g2 = _tc_mid(acc1, d, b1.reshape(1, HID), W2r)
    acc2 = _agg_kernel(g2, src_p, dst_p)
    g3 = _tc_mid(acc2, d, b2.reshape(1, HID), W21r)
    acc3 = _agg_kernel(g3, src_p, dst_p)
    g4 = _tc_mid(acc3, d, b21.reshape(1, HID), EYEr)
    acc4 = _agg_kernel(g4, src_p, dst_p)
    out = _tc_fin(acc4, d, W3, b3.reshape(1, 1))
    return out[:N]


# speedup vs baseline: 20.1764x; 2.4238x over previous
import functools

import jax
import jax.numpy as jnp
from jax import lax
from jax.experimental import pallas as pl
from jax.experimental.pallas import tpu as pltpu
from jax.experimental.pallas import tpu_sc as plsc

N = 100000
E = 1600000
FEATS = 4
HID = 32
HALF = 16

N_P = 100352
N_TRASH = 128
N_ACC = N_P + N_TRASH

CHUNK = 128
NSC = 2
NTILE = 16
NBUF = 4
E_PAD = 1605632

EPT = E_PAD // NTILE
NCHUNK = EPT // CHUNK
EPT_DEG = E_PAD // (NSC * NTILE)
NCHUNK_DEG = EPT_DEG // CHUNK

RPT = N_P // NTILE

R = 1024
NB = N_P // R

_MESH = plsc.VectorSubcoreMesh(core_axis_name="c", subcore_axis_name="s")
_SC_PARAMS = pltpu.CompilerParams(use_tc_tiling_on_sc=False)


@functools.partial(
    pl.kernel,
    out_type=jax.ShapeDtypeStruct((NSC, N_P), jnp.float32),
    mesh=_MESH,
    scratch_types=(
        [pltpu.VMEM((CHUNK,), jnp.int32) for _ in range(NBUF)]
        + [pltpu.VMEM((CHUNK,), jnp.float32)]
        + [pltpu.VMEM_SHARED((N_ACC,), jnp.float32)]
        + [pltpu.SemaphoreType.DMA] * (2 * NBUF)
    ),
    compiler_params=_SC_PARAMS,
)
def _deg_kernel(dst_hbm, init_hbm, ones_hbm, out_hbm, *sc):
    idxb = sc[0:NBUF]
    onesbuf = sc[NBUF]
    acc = sc[NBUF + 1]
    sem_i = sc[NBUF + 2:2 * NBUF + 2]
    sem_s = sc[2 * NBUF + 2:3 * NBUF + 2]
    c = lax.axis_index("c")
    s = lax.axis_index("s")
    r0 = s * RPT
    pltpu.sync_copy(init_hbm.at[c, pl.ds(r0, RPT)], acc.at[pl.ds(r0, RPT)])
    pltpu.sync_copy(ones_hbm, onesbuf)
    plsc.subcore_barrier()
    base = c * (E_PAD // NSC) + s * EPT_DEG

    def _load(b, j):
        pltpu.async_copy(dst_hbm.at[pl.ds(base + j * CHUNK, CHUNK)], idxb[b], sem_i[b])

    def _scat(b):
        pltpu.make_async_copy(dst_hbm.at[pl.ds(base, CHUNK)], idxb[b], sem_i[b]).wait()
        pltpu.async_copy(onesbuf, acc.at[idxb[b]], sem_s[b], add=True)

    def _drain(b):
        pltpu.make_async_copy(onesbuf, acc.at[idxb[b]], sem_s[b]).wait()

    for b in range(NBUF):
        _load(b, b)

    @pl.loop(0, NCHUNK_DEG - NBUF, step=NBUF)
    def _round(j0):
        for b in range(NBUF):
            _scat(b)
        for b in range(NBUF):
            _drain(b)
            _load(b, j0 + NBUF + b)

    for b in range(NBUF):
        _scat(b)
    for b in range(NBUF):
        _drain(b)
    plsc.subcore_barrier()
    pltpu.sync_copy(acc.at[pl.ds(r0, RPT)], out_hbm.at[c, pl.ds(r0, RPT)])


@functools.partial(
    pl.kernel,
    out_type=jax.ShapeDtypeStruct((NSC, N_P, HALF), jnp.float32),
    mesh=_MESH,
    scratch_types=(
        [pltpu.VMEM((2, CHUNK), jnp.int32) for _ in range(NBUF)]
        + [pltpu.VMEM((CHUNK, HALF), jnp.float32) for _ in range(NBUF)]
        + [pltpu.VMEM_SHARED((N_ACC, HALF), jnp.float32)]
        + [pltpu.SemaphoreType.DMA] * (3 * NBUF)
    ),
    compiler_params=_SC_PARAMS,
)
def _agg_kernel(g_hbm, esd_hbm, out_hbm, *sc):
    idxb = sc[0:NBUF]
    rows = sc[NBUF:2 * NBUF]
    acc = sc[2 * NBUF]
    sem_i = sc[2 * NBUF + 1:3 * NBUF + 1]
    sem_g = sc[3 * NBUF + 1:4 * NBUF + 1]
    sem_s = sc[4 * NBUF + 1:5 * NBUF + 1]
    c = lax.axis_index("c")
    s = lax.axis_index("s")
    gc = g_hbm.at[c]
    r0 = s * RPT
    pltpu.sync_copy(gc.at[pl.ds(r0, RPT)], acc.at[pl.ds(r0, RPT)])
    plsc.subcore_barrier()
    base = s * EPT

    def _load(b, j):
        pltpu.async_copy(
            esd_hbm.at[:, pl.ds(base + j * CHUNK, CHUNK)], idxb[b], sem_i[b])

    def _gather(b):
        pltpu.make_async_copy(
            esd_hbm.at[:, pl.ds(base, CHUNK)], idxb[b], sem_i[b]).wait()
        pltpu.async_copy(gc.at[idxb[b].at[0]], rows[b], sem_g[b])

    def _scat(b):
        pltpu.make_async_copy(gc.at[idxb[b].at[0]], rows[b], sem_g[b]).wait()
        pltpu.async_copy(rows[b], acc.at[idxb[b].at[1]], sem_s[b], add=True)

    def _drain(b):
        pltpu.make_async_copy(rows[b], acc.at[idxb[b].at[1]], sem_s[b]).wait()

    for b in range(NBUF):
        _load(b, b)

    @pl.loop(0, NCHUNK - NBUF, step=NBUF)
    def _round(j0):
        for b in range(NBUF):
            _gather(b)
        for b in range(NBUF):
            _scat(b)
        for b in range(NBUF):
            _drain(b)
            _load(b, j0 + NBUF + b)

    for b in range(NBUF):
        _gather(b)
    for b in range(NBUF):
        _scat(b)
    for b in range(NBUF):
        _drain(b)
    plsc.subcore_barrier()
    pltpu.sync_copy(acc.at[pl.ds(r0, RPT)], out_hbm.at[c, pl.ds(r0, RPT)])


def _tc0_body(deg_ref, x_ref, w_ref, d_ref, g_ref):
    dv = lax.rsqrt(deg_ref[0] + deg_ref[1])
    d_ref[...] = dv
    h = jnp.dot(x_ref[...], w_ref[0], preferred_element_type=jnp.float32)
    g_ref[0] = dv * h


def _tc_mid_body(acc_ref, d_ref, b_ref, w_ref, g_ref):
    dv = d_ref[...]
    a = jnp.concatenate([acc_ref[0], acc_ref[1]], axis=-1)
    xk = jnp.maximum(dv * a + b_ref[...], 0.0)
    g_ref[0] = dv * jnp.dot(xk, w_ref[0], preferred_element_type=jnp.float32)


def _tc_fin_body(acc_ref, d_ref, w_ref, b_ref, o_ref):
    dv = d_ref[...]
    a = jnp.concatenate([acc_ref[0], acc_ref[1]], axis=-1)
    o_ref[...] = dv * jnp.dot(a, w_ref[...], preferred_element_type=jnp.float32) + b_ref[...]


_tc0 = pl.pallas_call(
    _tc0_body,
    grid=(NSC, NB),
    in_specs=[
        pl.BlockSpec((NSC, R, 1), lambda c, i: (0, i, 0)),
        pl.BlockSpec((R, FEATS), lambda c, i: (i, 0)),
        pl.BlockSpec((1, FEATS, HALF), lambda c, i: (c, 0, 0)),
    ],
    out_specs=[
        pl.BlockSpec((R, 1), lambda c, i: (i, 0)),
        pl.BlockSpec((1, R, HALF), lambda c, i: (c, i, 0)),
    ],
    out_shape=[
        jax.ShapeDtypeStruct((N_P, 1), jnp.float32),
        jax.ShapeDtypeStruct((NSC, N_P, HALF), jnp.float32),
    ],
)

_tc_mid = pl.pallas_call(
    _tc_mid_body,
    grid=(NSC, NB),
    in_specs=[
        pl.BlockSpec((NSC, R, HALF), lambda c, i: (0, i, 0)),
        pl.BlockSpec((R, 1), lambda c, i: (i, 0)),
        pl.BlockSpec((1, HID), lambda c, i: (0, 0)),
        pl.BlockSpec((1, HID, HALF), lambda c, i: (c, 0, 0)),
    ],
    out_specs=pl.BlockSpec((1, R, HALF), lambda c, i: (c, i, 0)),
    out_shape=jax.ShapeDtypeStruct((NSC, N_P, HALF), jnp.float32),
)

_tc_fin = pl.pallas_call(
    _tc_fin_body,
    grid=(NB,),
    in_specs=[
        pl.BlockSpec((NSC, R, HALF), lambda i: (0, i, 0)),
        pl.BlockSpec((R, 1), lambda i: (i, 0)),
        pl.BlockSpec((HID, 1), lambda i: (0, 0)),
        pl.BlockSpec((1, 1), lambda i: (0, 0)),
    ],
    out_specs=pl.BlockSpec((R, 1), lambda i: (i, 0)),
    out_shape=jax.ShapeDtypeStruct((N_P, 1), jnp.float32),
)


def _split_w(w):
    return w.reshape(w.shape[0], NSC, HALF).transpose(1, 0, 2)


def kernel(x, edge_index, W1, b1, W2, b2, W21, b21, W3, b3):
    f32 = jnp.float32
    src = edge_index[0]
    dst = edge_index[1]
    pad = E_PAD - E
    src_p = jnp.concatenate([src, jnp.zeros((pad,), jnp.int32)])
    trash = N_P + (jnp.arange(pad, dtype=jnp.int32) % N_TRASH)
    dst_p = jnp.concatenate([dst, trash])
    esd = jnp.stack([src_p, dst_p])

    x_p = jnp.zeros((N_P, FEATS), f32).at[:N].set(x)
    deg_init = jnp.stack([jnp.ones((N_P,), f32), jnp.zeros((N_P,), f32)])
    ones_c = jnp.ones((CHUNK,), f32)

    W1r = _split_w(W1)
    W2r = _split_w(W2)
    W21r = _split_w(W21)
    EYEr = _split_w(jnp.eye(HID, dtype=f32))

    deg = _deg_kernel(dst_p, deg_init, ones_c)
    d, g1 = _tc0(deg.reshape(NSC, N_P, 1), x_p, W1r)
    acc1 = _agg_kernel(g1, esd)
    g2 = _tc_mid(acc1, d, b1.reshape(1, HID), W2r)
    acc2 = _agg_kernel(g2, esd)
    g3 = _tc_mid(acc2, d, b2.reshape(1, HID), W21r)
    acc3 = _agg_kernel(g3, esd)
    g4 = _tc_mid(acc3, d, b21.reshape(1, HID), EYEr)
    acc4 = _agg_kernel(g4, esd)
    out = _tc_fin(acc4, d, W3, b3.reshape(1, 1))
    return out[:N]


# speedup vs baseline: 22.9976x; 1.1398x over previous
import functools

import jax
import jax.numpy as jnp
from jax import lax
from jax.experimental import pallas as pl
from jax.experimental.pallas import tpu as pltpu
from jax.experimental.pallas import tpu_sc as plsc

N = 100000
E = 1600000
FEATS = 4
HID = 32
HALF = 16

N_P = 100352
N_TRASH = 128
N_ACC = N_P + N_TRASH

CHUNK = 256
CHUNK_DEG = 128
NSC = 2
NTILE = 16
NBUF = 4
E_PAD = 1605632

EPT = E_PAD // NTILE
NCHUNK = EPT // CHUNK
EPT_DEG = E_PAD // (NSC * NTILE)
NCHUNK_DEG = EPT_DEG // CHUNK_DEG

RPT = N_P // NTILE

R = 1024
NB = N_P // R

_MESH = plsc.VectorSubcoreMesh(core_axis_name="c", subcore_axis_name="s")
_SC_PARAMS = pltpu.CompilerParams(use_tc_tiling_on_sc=False)


@functools.partial(
    pl.kernel,
    out_type=jax.ShapeDtypeStruct((NSC, N_P), jnp.float32),
    mesh=_MESH,
    scratch_types=(
        [pltpu.VMEM((CHUNK_DEG,), jnp.int32) for _ in range(NBUF)]
        + [pltpu.VMEM((CHUNK_DEG,), jnp.float32)]
        + [pltpu.VMEM_SHARED((N_ACC,), jnp.float32)]
        + [pltpu.SemaphoreType.DMA] * (2 * NBUF)
    ),
    compiler_params=_SC_PARAMS,
)
def _deg_kernel(dst_hbm, init_hbm, ones_hbm, out_hbm, *sc):
    idxb = sc[0:NBUF]
    onesbuf = sc[NBUF]
    acc = sc[NBUF + 1]
    sem_i = sc[NBUF + 2:2 * NBUF + 2]
    sem_s = sc[2 * NBUF + 2:3 * NBUF + 2]
    c = lax.axis_index("c")
    s = lax.axis_index("s")
    r0 = s * RPT
    pltpu.sync_copy(init_hbm.at[c, pl.ds(r0, RPT)], acc.at[pl.ds(r0, RPT)])
    pltpu.sync_copy(ones_hbm, onesbuf)
    plsc.subcore_barrier()
    base = c * (E_PAD // NSC) + s * EPT_DEG

    def _load(b, j):
        pltpu.async_copy(dst_hbm.at[pl.ds(base + j * CHUNK_DEG, CHUNK_DEG)], idxb[b], sem_i[b])

    def _scat(b):
        pltpu.make_async_copy(dst_hbm.at[pl.ds(base, CHUNK_DEG)], idxb[b], sem_i[b]).wait()
        pltpu.async_copy(onesbuf, acc.at[idxb[b]], sem_s[b], add=True)

    def _drain(b):
        pltpu.make_async_copy(onesbuf, acc.at[idxb[b]], sem_s[b]).wait()

    for b in range(NBUF):
        _load(b, b)

    @pl.loop(0, NCHUNK_DEG - NBUF, step=NBUF)
    def _round(j0):
        for b in range(NBUF):
            _scat(b)
        for b in range(NBUF):
            _drain(b)
            _load(b, j0 + NBUF + b)

    for b in range(NBUF):
        _scat(b)
    for b in range(NBUF):
        _drain(b)
    plsc.subcore_barrier()
    pltpu.sync_copy(acc.at[pl.ds(r0, RPT)], out_hbm.at[c, pl.ds(r0, RPT)])


@functools.partial(
    pl.kernel,
    out_type=jax.ShapeDtypeStruct((NSC, N_P, HALF), jnp.float32),
    mesh=_MESH,
    scratch_types=(
        [pltpu.VMEM((2, CHUNK), jnp.int32) for _ in range(NBUF)]
        + [pltpu.VMEM((CHUNK, HALF), jnp.float32) for _ in range(NBUF)]
        + [pltpu.VMEM_SHARED((N_ACC, HALF), jnp.float32)]
        + [pltpu.SemaphoreType.DMA] * (3 * NBUF)
    ),
    compiler_params=_SC_PARAMS,
)
def _agg_kernel(g_hbm, esd_hbm, out_hbm, *sc):
    idxb = sc[0:NBUF]
    rows = sc[NBUF:2 * NBUF]
    acc = sc[2 * NBUF]
    sem_i = sc[2 * NBUF + 1:3 * NBUF + 1]
    sem_g = sc[3 * NBUF + 1:4 * NBUF + 1]
    sem_s = sc[4 * NBUF + 1:5 * NBUF + 1]
    c = lax.axis_index("c")
    s = lax.axis_index("s")
    gc = g_hbm.at[c]
    r0 = s * RPT
    pltpu.sync_copy(gc.at[pl.ds(r0, RPT)], acc.at[pl.ds(r0, RPT)])
    plsc.subcore_barrier()
    base = s * EPT

    def _load(b, j):
        pltpu.async_copy(
            esd_hbm.at[:, pl.ds(base + j * CHUNK, CHUNK)], idxb[b], sem_i[b])

    def _gather(b):
        pltpu.make_async_copy(
            esd_hbm.at[:, pl.ds(base, CHUNK)], idxb[b], sem_i[b]).wait()
        pltpu.async_copy(gc.at[idxb[b].at[0]], rows[b], sem_g[b])

    def _scat(b):
        pltpu.make_async_copy(gc.at[idxb[b].at[0]], rows[b], sem_g[b]).wait()
        pltpu.async_copy(rows[b], acc.at[idxb[b].at[1]], sem_s[b], add=True)

    def _drain(b):
        pltpu.make_async_copy(rows[b], acc.at[idxb[b].at[1]], sem_s[b]).wait()

    for b in range(NBUF):
        _load(b, b)

    @pl.loop(0, NCHUNK - NBUF, step=NBUF)
    def _round(j0):
        for b in range(NBUF):
            _gather(b)
        for b in range(NBUF):
            _scat(b)
        for b in range(NBUF):
            _drain(b)
            _load(b, j0 + NBUF + b)

    for b in range(NBUF):
        _gather(b)
    for b in range(NBUF):
        _scat(b)
    for b in range(NBUF):
        _drain(b)
    plsc.subcore_barrier()
    pltpu.sync_copy(acc.at[pl.ds(r0, RPT)], out_hbm.at[c, pl.ds(r0, RPT)])


def _tc0_body(deg_ref, x_ref, w_ref, d_ref, g_ref):
    dv = lax.rsqrt(deg_ref[0] + deg_ref[1])
    d_ref[...] = dv
    h = jnp.dot(x_ref[...], w_ref[0], preferred_element_type=jnp.float32)
    g_ref[0] = dv * h


def _tc_mid_body(acc_ref, d_ref, b_ref, w_ref, g_ref):
    dv = d_ref[...]
    a = jnp.concatenate([acc_ref[0], acc_ref[1]], axis=-1)
    xk = jnp.maximum(dv * a + b_ref[...], 0.0)
    g_ref[0] = dv * jnp.dot(xk, w_ref[0], preferred_element_type=jnp.float32)


def _tc_fin_body(acc_ref, d_ref, w_ref, b_ref, o_ref):
    dv = d_ref[...]
    a = jnp.concatenate([acc_ref[0], acc_ref[1]], axis=-1)
    o_ref[...] = dv * jnp.dot(a, w_ref[...], preferred_element_type=jnp.float32) + b_ref[...]


_tc0 = pl.pallas_call(
    _tc0_body,
    grid=(NSC, NB),
    in_specs=[
        pl.BlockSpec((NSC, R, 1), lambda c, i: (0, i, 0)),
        pl.BlockSpec((R, FEATS), lambda c, i: (i, 0)),
        pl.BlockSpec((1, FEATS, HALF), lambda c, i: (c, 0, 0)),
    ],
    out_specs=[
        pl.BlockSpec((R, 1), lambda c, i: (i, 0)),
        pl.BlockSpec((1, R, HALF), lambda c, i: (c, i, 0)),
    ],
    out_shape=[
        jax.ShapeDtypeStruct((N_P, 1), jnp.float32),
        jax.ShapeDtypeStruct((NSC, N_P, HALF), jnp.float32),
    ],
)

_tc_mid = pl.pallas_call(
    _tc_mid_body,
    grid=(NSC, NB),
    in_specs=[
        pl.BlockSpec((NSC, R, HALF), lambda c, i: (0, i, 0)),
        pl.BlockSpec((R, 1), lambda c, i: (i, 0)),
        pl.BlockSpec((1, HID), lambda c, i: (0, 0)),
        pl.BlockSpec((1, HID, HALF), lambda c, i: (c, 0, 0)),
    ],
    out_specs=pl.BlockSpec((1, R, HALF), lambda c, i: (c, i, 0)),
    out_shape=jax.ShapeDtypeStruct((NSC, N_P, HALF), jnp.float32),
)

_tc_fin = pl.pallas_call(
    _tc_fin_body,
    grid=(NB,),
    in_specs=[
        pl.BlockSpec((NSC, R, HALF), lambda i: (0, i, 0)),
        pl.BlockSpec((R, 1), lambda i: (i, 0)),
        pl.BlockSpec((HID, 1), lambda i: (0, 0)),
        pl.BlockSpec((1, 1), lambda i: (0, 0)),
    ],
    out_specs=pl.BlockSpec((R, 1), lambda i: (i, 0)),
    out_shape=jax.ShapeDtypeStruct((N_P, 1), jnp.float32),
)


def _split_w(w):
    return w.reshape(w.shape[0], NSC, HALF).transpose(1, 0, 2)


def kernel(x, edge_index, W1, b1, W2, b2, W21, b21, W3, b3):
    f32 = jnp.float32
    src = edge_index[0]
    dst = edge_index[1]
    pad = E_PAD - E
    src_p = jnp.concatenate([src, jnp.zeros((pad,), jnp.int32)])
    trash = N_P + (jnp.arange(pad, dtype=jnp.int32) % N_TRASH)
    dst_p = jnp.concatenate([dst, trash])
    esd = jnp.stack([src_p, dst_p])

    x_p = jnp.zeros((N_P, FEATS), f32).at[:N].set(x)
    deg_init = jnp.stack([jnp.ones((N_P,), f32), jnp.zeros((N_P,), f32)])
    ones_c = jnp.ones((CHUNK_DEG,), f32)

    W1r = _split_w(W1)
    W2r = _split_w(W2)
    W21r = _split_w(W21)
    EYEr = _split_w(jnp.eye(HID, dtype=f32))

    deg = _deg_kernel(dst_p, deg_init, ones_c)
    d, g1 = _tc0(deg.reshape(NSC, N_P, 1), x_p, W1r)
    acc1 = _agg_kernel(g1, esd)
    g2 = _tc_mid(acc1, d, b1.reshape(1, HID), W2r)
    acc2 = _agg_kernel(g2, esd)
    g3 = _tc_mid(acc2, d, b2.reshape(1, HID), W21r)
    acc3 = _agg_kernel(g3, esd)
    g4 = _tc_mid(acc3, d, b21.reshape(1, HID), EYEr)
    acc4 = _agg_kernel(g4, esd)
    out = _tc_fin(acc4, d, W3, b3.reshape(1, 1))
    return out[:N]


# speedup vs baseline: 41.4409x; 1.8020x over previous
import functools

import jax
import jax.numpy as jnp
from jax import lax
from jax.experimental import pallas as pl
from jax.experimental.pallas import tpu as pltpu
from jax.experimental.pallas import tpu_sc as plsc

N = 100000
E = 1600000
FEATS = 4
HID = 32
HALF = 16

N_P = 100352
N_TRASH = 128
N_ACC = N_P + N_TRASH

CHUNK = 256
CHUNK_DEG = 128
NSC = 2
NTILE = 16
NBUF = 4
E_PAD = 1605632

EPT = E_PAD // NTILE
NCHUNK = EPT // CHUNK
EPT_DEG = E_PAD // (NSC * NTILE)
NCHUNK_DEG = EPT_DEG // CHUNK_DEG

RPT = N_P // NTILE

R = 1024
NB = N_P // R

_MESH = plsc.VectorSubcoreMesh(core_axis_name="c", subcore_axis_name="s")
_SC_PARAMS = pltpu.CompilerParams(use_tc_tiling_on_sc=False)


@functools.partial(
    pl.kernel,
    out_type=jax.ShapeDtypeStruct((NSC, N_P), jnp.float32),
    mesh=_MESH,
    scratch_types=(
        [pltpu.VMEM((CHUNK_DEG,), jnp.int32) for _ in range(NBUF)]
        + [pltpu.VMEM((CHUNK_DEG,), jnp.float32)]
        + [pltpu.VMEM_SHARED((N_ACC,), jnp.float32)]
        + [pltpu.SemaphoreType.DMA] * (2 * NBUF)
    ),
    compiler_params=_SC_PARAMS,
)
def _deg_kernel(dst_hbm, init_hbm, ones_hbm, out_hbm, *sc):
    idxb = sc[0:NBUF]
    onesbuf = sc[NBUF]
    acc = sc[NBUF + 1]
    sem_i = sc[NBUF + 2:2 * NBUF + 2]
    sem_s = sc[2 * NBUF + 2:3 * NBUF + 2]
    c = lax.axis_index("c")
    s = lax.axis_index("s")
    r0 = s * RPT
    pltpu.sync_copy(init_hbm.at[c, pl.ds(r0, RPT)], acc.at[pl.ds(r0, RPT)])
    pltpu.sync_copy(ones_hbm, onesbuf)
    plsc.subcore_barrier()
    base = c * (E_PAD // NSC) + s * EPT_DEG

    def _load(b, j):
        pltpu.async_copy(dst_hbm.at[pl.ds(base + j * CHUNK_DEG, CHUNK_DEG)], idxb[b], sem_i[b])

    def _scat(b):
        pltpu.make_async_copy(dst_hbm.at[pl.ds(base, CHUNK_DEG)], idxb[b], sem_i[b]).wait()
        pltpu.async_copy(onesbuf, acc.at[idxb[b]], sem_s[b], add=True)

    def _drain(b):
        pltpu.make_async_copy(onesbuf, acc.at[idxb[b]], sem_s[b]).wait()

    for b in range(NBUF):
        _load(b, b)

    @pl.loop(0, NCHUNK_DEG - NBUF, step=NBUF)
    def _round(j0):
        for b in range(NBUF):
            _scat(b)
        for b in range(NBUF):
            _drain(b)
            _load(b, j0 + NBUF + b)

    for b in range(NBUF):
        _scat(b)
    for b in range(NBUF):
        _drain(b)
    plsc.subcore_barrier()
    pltpu.sync_copy(acc.at[pl.ds(r0, RPT)], out_hbm.at[c, pl.ds(r0, RPT)])


@functools.partial(
    pl.kernel,
    out_type=jax.ShapeDtypeStruct((NSC, N_P, HALF), jnp.float32),
    mesh=_MESH,
    scratch_types=(
        [pltpu.VMEM((2, CHUNK), jnp.int32) for _ in range(NBUF)]
        + [pltpu.VMEM((CHUNK, HALF), jnp.float32) for _ in range(NBUF)]
        + [pltpu.VMEM_SHARED((N_ACC, HALF), jnp.float32)]
        + [pltpu.SemaphoreType.DMA] * (3 * NBUF)
    ),
    compiler_params=_SC_PARAMS,
)
def _agg_kernel(g_hbm, esd_hbm, out_hbm, *sc):
    idxb = sc[0:NBUF]
    rows = sc[NBUF:2 * NBUF]
    acc = sc[2 * NBUF]
    sem_i = sc[2 * NBUF + 1:3 * NBUF + 1]
    sem_g = sc[3 * NBUF + 1:4 * NBUF + 1]
    sem_s = sc[4 * NBUF + 1:5 * NBUF + 1]
    c = lax.axis_index("c")
    s = lax.axis_index("s")
    gc = g_hbm.at[c]
    r0 = s * RPT
    pltpu.sync_copy(gc.at[pl.ds(r0, RPT)], acc.at[pl.ds(r0, RPT)])
    plsc.subcore_barrier()
    base = s * EPT

    def _load(b, j):
        pltpu.async_copy(
            esd_hbm.at[:, pl.ds(base + j * CHUNK, CHUNK)], idxb[b], sem_i[b])

    def _gather(b):
        pltpu.make_async_copy(
            esd_hbm.at[:, pl.ds(base, CHUNK)], idxb[b], sem_i[b]).wait()
        pltpu.async_copy(gc.at[idxb[b].at[0]], rows[b], sem_g[b])

    def _scat(b):
        pltpu.make_async_copy(gc.at[idxb[b].at[0]], rows[b], sem_g[b]).wait()
        pltpu.async_copy(rows[b], acc.at[idxb[b].at[1]], sem_s[b], add=True)

    def _drain(b):
        pltpu.make_async_copy(rows[b], acc.at[idxb[b].at[1]], sem_s[b]).wait()

    for b in range(NBUF):
        _load(b, b)

    @pl.loop(0, NCHUNK - NBUF, step=NBUF)
    def _round(j0):
        for b in range(NBUF):
            _gather(b)
        for b in range(NBUF):
            _scat(b)
        for b in range(NBUF):
            _drain(b)
            _load(b, j0 + NBUF + b)

    for b in range(NBUF):
        _gather(b)
    for b in range(NBUF):
        _scat(b)
    for b in range(NBUF):
        _drain(b)
    plsc.subcore_barrier()
    pltpu.sync_copy(acc.at[pl.ds(r0, RPT)], out_hbm.at[c, pl.ds(r0, RPT)])


NROW = N_P * HALF // 128
NROW4 = N_P * FEATS // 128
NROWD = N_P // 128
BRM = 256
NBM = NROW // BRM
BR0 = 448
NB0 = NROW4 // BR0
BRD = 112
NBD = NROWD // BRD


def _tc_d_body(deg_ref, d_ref):
    d_ref[...] = lax.rsqrt(deg_ref[0] + deg_ref[1])


def _tc0_body(x_ref, w_ref, dp_ref, g_ref):
    h = jnp.dot(x_ref[...], w_ref[0], preferred_element_type=jnp.float32)
    g_ref[0] = dp_ref[...] * h


def _tc_mid_body(acc_ref, dp_ref, bp_ref, w_ref, g_ref):
    dp = dp_ref[...]
    x0 = jnp.maximum(dp * acc_ref[0] + bp_ref[0], 0.0)
    x1 = jnp.maximum(dp * acc_ref[1] + bp_ref[1], 0.0)
    h = (jnp.dot(x0, w_ref[0, 0], preferred_element_type=jnp.float32)
         + jnp.dot(x1, w_ref[1, 0], preferred_element_type=jnp.float32))
    g_ref[0] = dp * h


def _tc4_body(acc_ref, dp_ref, bp_ref, g_ref):
    dp = dp_ref[...]
    g_ref[0] = dp * jnp.maximum(dp * acc_ref[0] + bp_ref[0, 0], 0.0)


def _tc_fin_body(acc_ref, dp_ref, w3t_ref, sum_ref, b3_ref, o_ref):
    z = acc_ref[0] * w3t_ref[0] + acc_ref[1] * w3t_ref[1]
    y = jnp.dot(z, sum_ref[...], preferred_element_type=jnp.float32)
    o_ref[...] = dp_ref[...] * y + b3_ref[...]


_tc_d = pl.pallas_call(
    _tc_d_body,
    grid=(NBD,),
    in_specs=[pl.BlockSpec((NSC, BRD, 128), lambda i: (0, i, 0))],
    out_specs=pl.BlockSpec((BRD, 128), lambda i: (i, 0)),
    out_shape=jax.ShapeDtypeStruct((NROWD, 128), jnp.float32),
)

_tc0 = pl.pallas_call(
    _tc0_body,
    grid=(NSC, NB0),
    in_specs=[
        pl.BlockSpec((BR0, 128), lambda c, i: (i, 0)),
        pl.BlockSpec((1, 128, 512), lambda c, i: (c, 0, 0)),
        pl.BlockSpec((BR0, 512), lambda c, i: (i, 0)),
    ],
    out_specs=pl.BlockSpec((1, BR0, 512), lambda c, i: (c, i, 0)),
    out_shape=jax.ShapeDtypeStruct((NSC, NROW4, 512), jnp.float32),
)

_tc_mid = pl.pallas_call(
    _tc_mid_body,
    grid=(NSC, NBM),
    in_specs=[
        pl.BlockSpec((NSC, BRM, 128), lambda c, i: (0, i, 0)),
        pl.BlockSpec((BRM, 128), lambda c, i: (i, 0)),
        pl.BlockSpec((NSC, 128), lambda c, i: (0, 0)),
        pl.BlockSpec((NSC, 1, 128, 128), lambda c, i: (0, c, 0, 0)),
    ],
    out_specs=pl.BlockSpec((1, BRM, 128), lambda c, i: (c, i, 0)),
    out_shape=jax.ShapeDtypeStruct((NSC, NROW, 128), jnp.float32),
)

_tc4 = pl.pallas_call(
    _tc4_body,
    grid=(NSC, NBM),
    in_specs=[
        pl.BlockSpec((1, BRM, 128), lambda c, i: (c, i, 0)),
        pl.BlockSpec((BRM, 128), lambda c, i: (i, 0)),
        pl.BlockSpec((1, 1, 128), lambda c, i: (c, 0, 0)),
    ],
    out_specs=pl.BlockSpec((1, BRM, 128), lambda c, i: (c, i, 0)),
    out_shape=jax.ShapeDtypeStruct((NSC, NROW, 128), jnp.float32),
)

_tc_fin = pl.pallas_call(
    _tc_fin_body,
    grid=(NBM,),
    in_specs=[
        pl.BlockSpec((NSC, BRM, 128), lambda i: (0, i, 0)),
        pl.BlockSpec((BRM, 128), lambda i: (i, 0)),
        pl.BlockSpec((NSC, 128), lambda i: (0, 0)),
        pl.BlockSpec((128, 128), lambda i: (0, 0)),
        pl.BlockSpec((1, 1), lambda i: (0, 0)),
    ],
    out_specs=pl.BlockSpec((BRM, 128), lambda i: (i, 0)),
    out_shape=jax.ShapeDtypeStruct((NROW, 128), jnp.float32),
)


def _bd(w):
    e8 = jnp.eye(8, dtype=jnp.float32)
    return jnp.stack([
        jnp.stack([jnp.kron(e8, w[16 * kh:16 * kh + 16, 16 * c:16 * c + 16])
                   for c in range(NSC)])
        for kh in range(NSC)])


def kernel(x, edge_index, W1, b1, W2, b2, W21, b21, W3, b3):
    f32 = jnp.float32
    src = edge_index[0]
    dst = edge_index[1]
    pad = E_PAD - E
    src_p = jnp.concatenate([src, jnp.zeros((pad,), jnp.int32)])
    trash = N_P + (jnp.arange(pad, dtype=jnp.int32) % N_TRASH)
    dst_p = jnp.concatenate([dst, trash])
    esd = jnp.stack([src_p, dst_p])

    x_pk = jnp.zeros((N_P, FEATS), f32).at[:N].set(x).reshape(NROW4, 128)
    deg_init = jnp.stack([jnp.ones((N_P,), f32), jnp.zeros((N_P,), f32)])
    ones_c = jnp.ones((CHUNK_DEG,), f32)

    e32 = jnp.eye(32, dtype=f32)
    W1big = jnp.stack([jnp.kron(e32, W1[:, 16 * c:16 * c + 16])
                       for c in range(NSC)])
    W2bd = _bd(W2)
    W21bd = _bd(W21)
    b1p = jnp.tile(b1.reshape(2, 16), (1, 8))
    b2p = jnp.tile(b2.reshape(2, 16), (1, 8))
    b21p = jnp.tile(b21.reshape(2, 16), (1, 8))
    w3t = jnp.tile(W3.reshape(2, 16), (1, 8))
    sumbd = jnp.kron(jnp.eye(8, dtype=f32), jnp.ones((16, 16), f32))

    deg = _deg_kernel(dst_p, deg_init, ones_c)
    d_flat = _tc_d(deg.reshape(NSC, NROWD, 128)).reshape(N_P)
    dp16 = jnp.repeat(d_flat, HALF).reshape(NROW, 128)

    g1 = _tc0(x_pk, W1big, dp16.reshape(NROW4, 512))
    acc1 = _agg_kernel(g1.reshape(NSC, N_P, HALF), esd)
    g2 = _tc_mid(acc1.reshape(NSC, NROW, 128), dp16, b1p, W2bd)
    acc2 = _agg_kernel(g2.reshape(NSC, N_P, HALF), esd)
    g3 = _tc_mid(acc2.reshape(NSC, NROW, 128), dp16, b2p, W21bd)
    acc3 = _agg_kernel(g3.reshape(NSC, N_P, HALF), esd)
    g4 = _tc4(acc3.reshape(NSC, NROW, 128), dp16, b21p.reshape(NSC, 1, 128))
    acc4 = _agg_kernel(g4.reshape(NSC, N_P, HALF), esd)
    out = _tc_fin(acc4.reshape(NSC, NROW, 128), dp16, w3t, sumbd,
                  b3.reshape(1, 1))
    return out.reshape(N_P, HALF)[:N, 0:1]
